# baseline (speedup 1.0000x reference)
import jax
import jax.numpy as jnp
from jax.experimental import pallas as pl
from jax.experimental.pallas import tpu as pltpu


def _prep_body(w1_ref, w2_ref, w3_ref, t1_ref, t2_ref, t3_ref):
    bf16 = jnp.bfloat16
    t1_ref[...] = jnp.zeros_like(t1_ref)
    t2_ref[...] = jnp.zeros_like(t2_ref)
    t3_ref[...] = jnp.zeros_like(t3_ref)
    s1 = w1_ref[0].astype(bf16)
    s2 = w2_ref[...].astype(bf16)
    s3 = w3_ref[...].astype(bf16)
    for wo in range(24):
        t1_ref[0, wo:wo + 5, wo * 32:(wo + 1) * 32] = s1
    for wo in range(20):
        t2_ref[0, wo * 32:wo * 32 + 160, wo * 32:(wo + 1) * 32] = s2
    for wo in range(16):
        t3_ref[0, wo * 32:wo * 32 + 160, wo * 64:(wo + 1) * 64] = s3


def _build_toeplitz(conv1_w, conv2_w, conv3_w):
    bf16 = jnp.bfloat16
    t1, t2, t3 = pl.pallas_call(
        _prep_body,
        out_shape=(
            jax.ShapeDtypeStruct((5, 28, 768), bf16),
            jax.ShapeDtypeStruct((5, 768, 640), bf16),
            jax.ShapeDtypeStruct((5, 640, 1024), bf16),
        ),
        grid=(5,),
        in_specs=[
            pl.BlockSpec((1, 5, 32), lambda i: (i, 0, 0)),
            pl.BlockSpec((160, 32), lambda i: (i, 0)),
            pl.BlockSpec((160, 64), lambda i: (i, 0)),
        ],
        out_specs=(
            pl.BlockSpec((1, 28, 768), lambda i: (i, 0, 0)),
            pl.BlockSpec((1, 768, 640), lambda i: (i, 0, 0)),
            pl.BlockSpec((1, 640, 1024), lambda i: (i, 0, 0)),
        ),
        compiler_params=pltpu.CompilerParams(
            dimension_semantics=("parallel",),
        ),
    )(conv1_w.reshape(5, 5, 32), conv2_w, conv3_w)
    return t1.reshape(140, 768), t2, t3


def _fused_body(x_ref, t1_ref, b1_ref, t2_ref, b2_ref, t3_ref, b3_ref,
                w1_ref, fb1_ref, w2_ref, fb2_ref, o_ref):
    bb = x_ref.shape[1]
    f32 = jnp.float32
    bf16 = jnp.bfloat16

    x = x_ref[...].reshape(28 * bb, 28)
    x5 = jnp.concatenate(
        [x[di * bb:(di + 24) * bb, :] for di in range(5)], axis=1)
    y1 = jnp.maximum(
        jnp.dot(x5, t1_ref[...], preferred_element_type=f32) + b1_ref[...],
        0.0).astype(bf16)

    xc2 = jnp.concatenate(
        [y1[di * bb:(di + 20) * bb, :] for di in range(5)], axis=1)
    y2 = jnp.maximum(
        jnp.dot(xc2, t2_ref[...], preferred_element_type=f32) + b2_ref[...],
        0.0).astype(bf16)

    xc3 = jnp.concatenate(
        [y2[di * bb:(di + 16) * bb, :] for di in range(5)], axis=1)
    y3 = jnp.maximum(
        jnp.dot(xc3, t3_ref[...], preferred_element_type=f32) + b3_ref[...],
        0.0).astype(bf16)

    y3f = y3.astype(f32)
    acc = jnp.dot(y3f[0:bb, :], w1_ref[0], preferred_element_type=f32)
    for h in range(1, 16):
        acc = acc + jnp.dot(y3f[h * bb:(h + 1) * bb, :], w1_ref[h],
                            preferred_element_type=f32)
    h1 = jnp.maximum(acc + fb1_ref[...], 0.0)

    logits = (jnp.dot(h1, w2_ref[...], preferred_element_type=f32)
              + fb2_ref[...])
    m = jnp.max(logits, axis=-1, keepdims=True)
    s = logits - m
    lse = jnp.log(jnp.sum(jnp.exp(s), axis=-1, keepdims=True))
    o_ref[...] = (s - lse).astype(o_ref.dtype)


def kernel(x, conv1_w, conv1_b, conv2_w, conv2_b, conv3_w, conv3_b,
           fc1_w, fc1_b, fc2_w, fc2_b):
    B = x.shape[0]
    bb = 64

    t1, t2, t3 = _build_toeplitz(conv1_w, conv2_w, conv3_w)
    t2 = t2.reshape(5 * 768, 640)
    t3 = t3.reshape(5 * 640, 1024)
    b1t = jnp.tile(conv1_b, (1, 24))
    b2t = jnp.tile(conv2_b, (1, 20))
    b3t = jnp.tile(conv3_b, (1, 16))
    w1r = fc1_w.reshape(16, 1024, 256)
    w2b = fc2_w
    xr = (x.reshape(B, 28, 28).transpose(1, 0, 2)
          .astype(jnp.bfloat16))

    full2 = lambda a: pl.BlockSpec(a.shape, lambda i: (0,) * a.ndim)
    return pl.pallas_call(
        _fused_body,
        out_shape=jax.ShapeDtypeStruct((B, 10), jnp.float32),
        grid=(B // bb,),
        in_specs=[
            pl.BlockSpec((28, bb, 28), lambda i: (0, i, 0)),
            full2(t1), full2(b1t), full2(t2), full2(b2t),
            full2(t3), full2(b3t), full2(w1r), full2(fc1_b),
            full2(w2b), full2(fc2_b),
        ],
        out_specs=pl.BlockSpec((bb, 10), lambda i: (i, 0)),
        compiler_params=pltpu.CompilerParams(
            dimension_semantics=("parallel",),
            vmem_limit_bytes=100 * 1024 * 1024,
        ),
    )(xr, t1, b1t, t2, b2t, t3, b3t, w1r, fc1_b, w2b, fc2_b)


# speedup vs baseline: 1.0322x; 1.0322x over previous
import jax
import jax.numpy as jnp
from jax.experimental import pallas as pl
from jax.experimental.pallas import tpu as pltpu


def _prep_body(w1_ref, w2_ref, w3_ref, b1_ref, b2_ref, b3_ref,
               t1_ref, t2_ref, t3_ref, b1t_ref, b2t_ref, b3t_ref):
    bf16 = jnp.bfloat16
    t1_ref[...] = jnp.zeros_like(t1_ref)
    t2_ref[...] = jnp.zeros_like(t2_ref)
    t3_ref[...] = jnp.zeros_like(t3_ref)
    s1 = w1_ref[0].astype(bf16)
    s2 = w2_ref[...].astype(bf16)
    s3 = w3_ref[...].astype(bf16)
    for wo in range(24):
        t1_ref[0, wo:wo + 5, wo * 32:(wo + 1) * 32] = s1
    for wo in range(20):
        t2_ref[0, wo * 32:wo * 32 + 160, wo * 32:(wo + 1) * 32] = s2
    for wo in range(16):
        t3_ref[0, wo * 32:wo * 32 + 160, wo * 64:(wo + 1) * 64] = s3
    for wo in range(24):
        b1t_ref[0:1, wo * 32:(wo + 1) * 32] = b1_ref[...]
    for wo in range(20):
        b2t_ref[0:1, wo * 32:(wo + 1) * 32] = b2_ref[...]
    for wo in range(16):
        b3t_ref[0:1, wo * 64:(wo + 1) * 64] = b3_ref[...]


def _build_toeplitz(conv1_w, conv2_w, conv3_w, conv1_b, conv2_b, conv3_b):
    bf16 = jnp.bfloat16
    f32 = jnp.float32
    t1, t2, t3, b1t, b2t, b3t = pl.pallas_call(
        _prep_body,
        out_shape=(
            jax.ShapeDtypeStruct((5, 28, 768), bf16),
            jax.ShapeDtypeStruct((5, 768, 640), bf16),
            jax.ShapeDtypeStruct((5, 640, 1024), bf16),
            jax.ShapeDtypeStruct((1, 768), f32),
            jax.ShapeDtypeStruct((1, 640), f32),
            jax.ShapeDtypeStruct((1, 1024), f32),
        ),
        grid=(5,),
        in_specs=[
            pl.BlockSpec((1, 5, 32), lambda i: (i, 0, 0)),
            pl.BlockSpec((160, 32), lambda i: (i, 0)),
            pl.BlockSpec((160, 64), lambda i: (i, 0)),
            pl.BlockSpec((1, 32), lambda i: (0, 0)),
            pl.BlockSpec((1, 32), lambda i: (0, 0)),
            pl.BlockSpec((1, 64), lambda i: (0, 0)),
        ],
        out_specs=(
            pl.BlockSpec((1, 28, 768), lambda i: (i, 0, 0)),
            pl.BlockSpec((1, 768, 640), lambda i: (i, 0, 0)),
            pl.BlockSpec((1, 640, 1024), lambda i: (i, 0, 0)),
            pl.BlockSpec((1, 768), lambda i: (0, 0)),
            pl.BlockSpec((1, 640), lambda i: (0, 0)),
            pl.BlockSpec((1, 1024), lambda i: (0, 0)),
        ),
        compiler_params=pltpu.CompilerParams(
            dimension_semantics=("arbitrary",),
        ),
    )(conv1_w.reshape(5, 5, 32), conv2_w, conv3_w, conv1_b, conv2_b, conv3_b)
    return t1.reshape(140, 768), t2, t3, b1t, b2t, b3t


def _chain(x, t1_ref, b1_ref, t2_ref, b2_ref, t3_ref, b3_ref,
           w1_ref, fb1_ref, w2_ref, fb2_ref):
    f32 = jnp.float32
    bf16 = jnp.bfloat16
    bb = x.shape[0] // 28

    x5 = jnp.concatenate(
        [x[di * bb:(di + 24) * bb, :] for di in range(5)], axis=1)
    y1 = jnp.maximum(
        jnp.dot(x5, t1_ref[...], preferred_element_type=f32) + b1_ref[...],
        0.0).astype(bf16)

    xc2 = jnp.concatenate(
        [y1[di * bb:(di + 20) * bb, :] for di in range(5)], axis=1)
    y2 = jnp.maximum(
        jnp.dot(xc2, t2_ref[...], preferred_element_type=f32) + b2_ref[...],
        0.0).astype(bf16)

    xc3 = jnp.concatenate(
        [y2[di * bb:(di + 16) * bb, :] for di in range(5)], axis=1)
    y3 = jnp.maximum(
        jnp.dot(xc3, t3_ref[...], preferred_element_type=f32) + b3_ref[...],
        0.0).astype(bf16)

    y3f = y3.astype(f32)
    acc = jnp.dot(y3f[0:bb, :], w1_ref[0], preferred_element_type=f32)
    for h in range(1, 16):
        acc = acc + jnp.dot(y3f[h * bb:(h + 1) * bb, :], w1_ref[h],
                            preferred_element_type=f32)
    h1 = jnp.maximum(acc + fb1_ref[...], 0.0)

    logits = (jnp.dot(h1, w2_ref[...], preferred_element_type=f32)
              + fb2_ref[...])
    m = jnp.max(logits, axis=-1, keepdims=True)
    s = logits - m
    lse = jnp.log(jnp.sum(jnp.exp(s), axis=-1, keepdims=True))
    return s - lse


def _fused_body(x_ref, t1_ref, b1_ref, t2_ref, b2_ref, t3_ref, b3_ref,
                w1_ref, fb1_ref, w2_ref, fb2_ref, o_ref):
    bb = x_ref.shape[1]
    xs = x_ref[...].reshape(28 * bb, 28)
    o_ref[...] = _chain(
        xs, t1_ref, b1_ref, t2_ref, b2_ref, t3_ref, b3_ref,
        w1_ref, fb1_ref, w2_ref, fb2_ref).astype(o_ref.dtype)


def kernel(x, conv1_w, conv1_b, conv2_w, conv2_b, conv3_w, conv3_b,
           fc1_w, fc1_b, fc2_w, fc2_b):
    B = x.shape[0]
    bb = 64

    t1, t2, t3, b1t, b2t, b3t = _build_toeplitz(
        conv1_w, conv2_w, conv3_w, conv1_b, conv2_b, conv3_b)
    t2 = t2.reshape(5 * 768, 640)
    t3 = t3.reshape(5 * 640, 1024)
    w1r = fc1_w.reshape(16, 1024, 256)
    w2b = fc2_w
    xr = (x.reshape(B, 28, 28).transpose(1, 0, 2)
          .astype(jnp.bfloat16))

    full2 = lambda a: pl.BlockSpec(a.shape, lambda i: (0,) * a.ndim)
    return pl.pallas_call(
        _fused_body,
        out_shape=jax.ShapeDtypeStruct((B, 10), jnp.float32),
        grid=(B // bb,),
        in_specs=[
            pl.BlockSpec((28, bb, 28), lambda i: (0, i, 0)),
            full2(t1), full2(b1t), full2(t2), full2(b2t),
            full2(t3), full2(b3t), full2(w1r), full2(fc1_b),
            full2(w2b), full2(fc2_b),
        ],
        out_specs=pl.BlockSpec((bb, 10), lambda i: (i, 0)),
        compiler_params=pltpu.CompilerParams(
            dimension_semantics=("parallel",),
            vmem_limit_bytes=100 * 1024 * 1024,
        ),
    )(xr, t1, b1t, t2, b2t, t3, b3t, w1r, fc1_b, w2b, fc2_b)


# speedup vs baseline: 1.0389x; 1.0065x over previous
import jax
import jax.numpy as jnp
from jax.experimental import pallas as pl
from jax.experimental.pallas import tpu as pltpu


def _prep_body(w1_ref, w2_ref, w3_ref, b1_ref, b2_ref, b3_ref,
               t1_ref, t2_ref, t3_ref, b1t_ref, b2t_ref, b3t_ref):
    bf16 = jnp.bfloat16
    t1_ref[...] = jnp.zeros_like(t1_ref)
    t2_ref[...] = jnp.zeros_like(t2_ref)
    t3_ref[...] = jnp.zeros_like(t3_ref)
    s1 = w1_ref[0].astype(bf16)
    s2 = w2_ref[...].astype(bf16)
    s3 = w3_ref[...].astype(bf16)
    for wo in range(24):
        t1_ref[0, wo:wo + 5, wo * 32:(wo + 1) * 32] = s1
    for wo in range(20):
        t2_ref[0, wo * 32:wo * 32 + 160, wo * 32:(wo + 1) * 32] = s2
    for wo in range(16):
        t3_ref[0, wo * 32:wo * 32 + 160, wo * 64:(wo + 1) * 64] = s3
    for wo in range(24):
        b1t_ref[0:1, wo * 32:(wo + 1) * 32] = b1_ref[...]
    for wo in range(20):
        b2t_ref[0:1, wo * 32:(wo + 1) * 32] = b2_ref[...]
    for wo in range(16):
        b3t_ref[0:1, wo * 64:(wo + 1) * 64] = b3_ref[...]


def _build_toeplitz(conv1_w, conv2_w, conv3_w, conv1_b, conv2_b, conv3_b):
    bf16 = jnp.bfloat16
    f32 = jnp.float32
    t1, t2, t3, b1t, b2t, b3t = pl.pallas_call(
        _prep_body,
        out_shape=(
            jax.ShapeDtypeStruct((5, 28, 768), bf16),
            jax.ShapeDtypeStruct((5, 768, 640), bf16),
            jax.ShapeDtypeStruct((5, 640, 1024), bf16),
            jax.ShapeDtypeStruct((1, 768), f32),
            jax.ShapeDtypeStruct((1, 640), f32),
            jax.ShapeDtypeStruct((1, 1024), f32),
        ),
        grid=(5,),
        in_specs=[
            pl.BlockSpec((1, 5, 32), lambda i: (i, 0, 0)),
            pl.BlockSpec((160, 32), lambda i: (i, 0)),
            pl.BlockSpec((160, 64), lambda i: (i, 0)),
            pl.BlockSpec((1, 32), lambda i: (0, 0)),
            pl.BlockSpec((1, 32), lambda i: (0, 0)),
            pl.BlockSpec((1, 64), lambda i: (0, 0)),
        ],
        out_specs=(
            pl.BlockSpec((1, 28, 768), lambda i: (i, 0, 0)),
            pl.BlockSpec((1, 768, 640), lambda i: (i, 0, 0)),
            pl.BlockSpec((1, 640, 1024), lambda i: (i, 0, 0)),
            pl.BlockSpec((1, 768), lambda i: (0, 0)),
            pl.BlockSpec((1, 640), lambda i: (0, 0)),
            pl.BlockSpec((1, 1024), lambda i: (0, 0)),
        ),
        compiler_params=pltpu.CompilerParams(
            dimension_semantics=("arbitrary",),
        ),
    )(conv1_w.reshape(5, 5, 32), conv2_w, conv3_w, conv1_b, conv2_b, conv3_b)
    return t1.reshape(140, 768), t2, t3, b1t, b2t, b3t


def _chain(x, t1_ref, b1_ref, t2_ref, b2_ref, t3_ref, b3_ref,
           w1_ref, fb1_ref, w2_ref, fb2_ref):
    f32 = jnp.float32
    bf16 = jnp.bfloat16
    bb = x.shape[0] // 28

    x5 = jnp.concatenate(
        [x[di * bb:(di + 24) * bb, :] for di in range(5)], axis=1)
    y1 = jnp.maximum(
        jnp.dot(x5, t1_ref[...], preferred_element_type=f32) + b1_ref[...],
        0.0).astype(bf16)

    xc2 = jnp.concatenate(
        [y1[di * bb:(di + 20) * bb, :] for di in range(5)], axis=1)
    y2 = jnp.maximum(
        jnp.dot(xc2, t2_ref[...], preferred_element_type=f32) + b2_ref[...],
        0.0).astype(bf16)

    xc3 = jnp.concatenate(
        [y2[di * bb:(di + 16) * bb, :] for di in range(5)], axis=1)
    y3 = jnp.maximum(
        jnp.dot(xc3, t3_ref[...], preferred_element_type=f32) + b3_ref[...],
        0.0).astype(bf16)

    y3f = y3.astype(f32)
    acc = jnp.dot(y3f[0:bb, :], w1_ref[0], preferred_element_type=f32)
    for h in range(1, 16):
        acc = acc + jnp.dot(y3f[h * bb:(h + 1) * bb, :], w1_ref[h],
                            preferred_element_type=f32)
    h1 = jnp.maximum(acc + fb1_ref[...], 0.0)

    logits = (jnp.dot(h1, w2_ref[...], preferred_element_type=f32)
              + fb2_ref[...])
    m = jnp.max(logits, axis=-1, keepdims=True)
    s = logits - m
    lse = jnp.log(jnp.sum(jnp.exp(s), axis=-1, keepdims=True))
    return s - lse


def _fused_body(x_ref, t1_ref, b1_ref, t2_ref, b2_ref, t3_ref, b3_ref,
                w1_ref, fb1_ref, w2_ref, fb2_ref, o_ref):
    bb = x_ref.shape[0]
    xs = (x_ref[...].astype(jnp.bfloat16).transpose(1, 0, 2)
          .reshape(28 * bb, 28))
    o_ref[...] = _chain(
        xs, t1_ref, b1_ref, t2_ref, b2_ref, t3_ref, b3_ref,
        w1_ref, fb1_ref, w2_ref, fb2_ref).astype(o_ref.dtype)


def kernel(x, conv1_w, conv1_b, conv2_w, conv2_b, conv3_w, conv3_b,
           fc1_w, fc1_b, fc2_w, fc2_b):
    B = x.shape[0]
    bb = 64

    t1, t2, t3, b1t, b2t, b3t = _build_toeplitz(
        conv1_w, conv2_w, conv3_w, conv1_b, conv2_b, conv3_b)
    t2 = t2.reshape(5 * 768, 640)
    t3 = t3.reshape(5 * 640, 1024)
    w1r = fc1_w.reshape(16, 1024, 256)
    w2b = fc2_w
    xr = x.reshape(B, 28, 28)

    full2 = lambda a: pl.BlockSpec(a.shape, lambda i: (0,) * a.ndim)
    return pl.pallas_call(
        _fused_body,
        out_shape=jax.ShapeDtypeStruct((B, 10), jnp.float32),
        grid=(B // bb,),
        in_specs=[
            pl.BlockSpec((bb, 28, 28), lambda i: (i, 0, 0)),
            full2(t1), full2(b1t), full2(t2), full2(b2t),
            full2(t3), full2(b3t), full2(w1r), full2(fc1_b),
            full2(w2b), full2(fc2_b),
        ],
        out_specs=pl.BlockSpec((bb, 10), lambda i: (i, 0)),
        compiler_params=pltpu.CompilerParams(
            dimension_semantics=("parallel",),
            vmem_limit_bytes=100 * 1024 * 1024,
        ),
    )(xr, t1, b1t, t2, b2t, t3, b3t, w1r, fc1_b, w2b, fc2_b)
